# baseline (speedup 1.0000x reference)
import functools

import jax
import jax.numpy as jnp
from jax.experimental import pallas as pl
from jax.experimental.pallas import tpu as pltpu
from jax.experimental.pallas import tpu_sc as plsc

_E = 8
_D = 2048
_DFF = 1408
_SFF = 5632

_TILE = 512
_MAXT = 16
_MAXR = _MAXT * _TILE
_CHK = 512
_SCH = 256
_TCH = 512
_GW = 128
_SUB = 256


def _routing_body(x_ref, gw_ref, p1_ref, p2_ref, w1_ref, w2_ref, te_ref,
                  tv_ref):
    x = x_ref[...]
    gw = gw_ref[...]
    t = x.shape[0]
    logits = jax.lax.dot_general(
        x, gw, (((1,), (1,)), ((), ())), preferred_element_type=jnp.float32)
    iota = jax.lax.broadcasted_iota(jnp.int32, logits.shape, 1)
    m1 = jnp.max(logits, axis=1, keepdims=True)
    i1 = jnp.min(jnp.where(logits == m1, iota, _E), axis=1, keepdims=True)
    oh1 = iota == i1
    l2 = jnp.where(oh1, -jnp.inf, logits)
    m2 = jnp.max(l2, axis=1, keepdims=True)
    i2 = jnp.min(jnp.where(l2 == m2, iota, _E), axis=1, keepdims=True)
    oh2 = iota == i2
    w1_ref[...] = 1.0 / (1.0 + jnp.exp(m2 - m1))
    w2_ref[...] = 1.0 - w1_ref[...]

    o1 = oh1.astype(jnp.float32)
    o2 = oh2.astype(jnp.float32)
    li = jax.lax.broadcasted_iota(jnp.int32, (_CHK, _CHK), 0)
    lj = jax.lax.broadcasted_iota(jnp.int32, (_CHK, _CHK), 1)
    ltri = (li > lj).astype(jnp.float32)
    ranks = []
    carry = jnp.zeros((1, _E), jnp.float32)
    for o, oh in ((o1, oh1), (o2, oh2)):
        rk = []
        for c in range(t // _CHK):
            oc = o[c * _CHK:(c + 1) * _CHK, :]
            pre = jax.lax.dot_general(
                ltri, oc, (((1,), (0,)), ((), ())),
                preferred_element_type=jnp.float32) + carry
            ohc = oh[c * _CHK:(c + 1) * _CHK, :]
            rk.append(jnp.sum(jnp.where(ohc, pre, 0.0), axis=1,
                              keepdims=True))
            carry = carry + jnp.sum(oc, axis=0, keepdims=True)
        ranks.append(jnp.concatenate(rk, axis=0))
    counts = carry
    padded = jnp.ceil(counts / _TILE) * _TILE
    ei = jax.lax.broadcasted_iota(jnp.int32, (_E, _E), 0)
    ej = jax.lax.broadcasted_iota(jnp.int32, (_E, _E), 1)
    etri = (ei < ej).astype(jnp.float32)
    padstart = jax.lax.dot_general(
        padded, etri, (((1,), (0,)), ((), ())),
        preferred_element_type=jnp.float32)
    start1 = jnp.sum(jnp.where(oh1, padstart, 0.0), axis=1, keepdims=True)
    start2 = jnp.sum(jnp.where(oh2, padstart, 0.0), axis=1, keepdims=True)
    p1_ref[...] = (start1 + ranks[0]).astype(jnp.int32)
    p2_ref[...] = (start2 + ranks[1]).astype(jnp.int32)

    padend = (padstart + padded).astype(jnp.int32)
    base = jax.lax.broadcasted_iota(jnp.int32, (_E, _MAXT), 1) * _TILE
    ends = jnp.swapaxes(padend, 0, 1)
    texp = jnp.sum((base >= ends).astype(jnp.int32), axis=0, keepdims=True)
    te_ref[...] = jnp.minimum(texp, _E - 1)
    total = jnp.sum(padded).astype(jnp.int32)
    tbase = jax.lax.broadcasted_iota(jnp.int32, (1, _MAXT), 1) * _TILE
    tv_ref[...] = (tbase < total).astype(jnp.int32)


def _gu_body(te_ref, tv_ref, xg_ref, ge_ref, ue_ref, h_ref):
    j = pl.program_id(0)

    @pl.when(tv_ref[j] > 0)
    def _():
        xg = xg_ref[...]
        ge = ge_ref[0].astype(jnp.bfloat16)
        ue = ue_ref[0].astype(jnp.bfloat16)
        g = jax.lax.dot_general(
            xg, ge, (((1,), (1,)), ((), ())),
            preferred_element_type=jnp.float32)
        u = jax.lax.dot_general(
            xg, ue, (((1,), (1,)), ((), ())),
            preferred_element_type=jnp.float32)
        h_ref[...] = (g * jax.nn.sigmoid(g) * u).astype(jnp.bfloat16)


def _dn_body(te_ref, tv_ref, h_ref, de_ref, y_ref):
    j = pl.program_id(0)

    @pl.when(tv_ref[j] > 0)
    def _():
        h = h_ref[...]
        de = de_ref[0].astype(jnp.bfloat16)
        y_ref[...] = jax.lax.dot_general(
            h, de, (((1,), (1,)), ((), ())),
            preferred_element_type=jnp.float32).astype(jnp.bfloat16)


def _combine_body(y1_ref, y2_ref, w1_ref, w2_ref, sh_ref, out_ref):
    y1 = y1_ref[...].astype(jnp.float32)
    y2 = y2_ref[...].astype(jnp.float32)
    out_ref[...] = y1 * w1_ref[...] + y2 * w2_ref[...] + sh_ref[...]


def _shared_body(xb_ref, wg_ref, wu_ref, wd_ref, wsg_ref, out_ref, sig_ref):
    s = pl.program_id(0)
    xb = xb_ref[...]

    @pl.when(s == 0)
    def _():
        xf = xb.astype(jnp.float32)
        logit = jnp.sum(xf * wsg_ref[...], axis=1, keepdims=True)
        sig_ref[...] = jax.nn.sigmoid(logit)

    wg = wg_ref[...].astype(jnp.bfloat16)
    wu = wu_ref[...].astype(jnp.bfloat16)
    wd = wd_ref[...].astype(jnp.bfloat16)
    g = jax.lax.dot_general(
        xb, wg, (((1,), (1,)), ((), ())), preferred_element_type=jnp.float32)
    u = jax.lax.dot_general(
        xb, wu, (((1,), (1,)), ((), ())), preferred_element_type=jnp.float32)
    h = (g * jax.nn.sigmoid(g) * u * sig_ref[...]).astype(jnp.bfloat16)
    for k in range(4):
        hk = h[k * _TCH:(k + 1) * _TCH, :]
        tmp = jax.lax.dot_general(
            hk, wd, (((1,), (1,)), ((), ())),
            preferred_element_type=jnp.float32)

        @pl.when(s == 0)
        def _():
            out_ref[k * _TCH:(k + 1) * _TCH, :] = tmp

        @pl.when(s > 0)
        def _():
            out_ref[k * _TCH:(k + 1) * _TCH, :] += tmp


def _expand_subrow_idx(idx, nsub):
    n = idx.shape[1]
    return (idx.reshape(n, 1) * nsub
            + jnp.arange(nsub, dtype=jnp.int32).reshape(1, nsub)
            ).reshape(1, n * nsub)


def _sc_scatter_rows(rows_src, idx, out_rows):
    n, d = rows_src.shape
    nsub = d // _SUB
    rs = rows_src.reshape(n * nsub, _SUB)
    idx8 = _expand_subrow_idx(idx, nsub)
    mesh = plsc.VectorSubcoreMesh(core_axis_name="core",
                                  subcore_axis_name="subcore")

    @functools.partial(
        pl.kernel,
        out_type=jax.ShapeDtypeStruct((out_rows * nsub, _SUB),
                                      rows_src.dtype),
        mesh=mesh)
    def k(x_hbm, i_hbm, o_hbm):
        def body(x_vmem, i_vmem):
            pltpu.sync_copy(x_vmem, o_hbm.at[i_vmem.at[0]])

        pltpu.emit_pipeline(
            body,
            grid=(n * nsub // _GW,),
            in_specs=[pl.BlockSpec((_GW, _SUB), lambda i: (i, 0)),
                      pl.BlockSpec((1, _GW), lambda i: (0, i))],
            out_specs=[],
            core_axis_name='subcore',
            dimension_semantics=(pltpu.PARALLEL,),
        )(x_hbm, i_hbm)

    return k(rs, idx8).reshape(out_rows, d)


def _sc_gather_rows(src, idx):
    n = idx.shape[1]
    r, d = src.shape
    nsub = d // _SUB
    srcs = src.reshape(r * nsub, _SUB)
    idx8 = _expand_subrow_idx(idx, nsub)
    mesh = plsc.VectorSubcoreMesh(core_axis_name="core",
                                  subcore_axis_name="subcore")

    @functools.partial(
        pl.kernel,
        out_type=jax.ShapeDtypeStruct((n * nsub, _SUB), src.dtype),
        mesh=mesh)
    def k(x_hbm, i_hbm, o_hbm):
        def body(i_vmem, o_vmem):
            pltpu.sync_copy(x_hbm.at[i_vmem.at[0]], o_vmem)

        pltpu.emit_pipeline(
            body,
            grid=(n * nsub // _GW,),
            in_specs=[pl.BlockSpec((1, _GW), lambda i: (0, i))],
            out_specs=[pl.BlockSpec((_GW, _SUB), lambda i: (i, 0))],
            core_axis_name='subcore',
            dimension_semantics=(pltpu.PARALLEL,),
        )(i_hbm, o_hbm)

    return k(srcs, idx8).reshape(n, d)


def kernel(hidden_states, gate_w, expert_gate_w, expert_up_w, expert_down_w,
           shared_gate_w, shared_up_w, shared_down_w, shared_expert_gate_w):
    b, seq, d = hidden_states.shape
    t = b * seq
    x = hidden_states.reshape(t, d)
    xb = x.astype(jnp.bfloat16)

    n_s = _SFF // _SCH
    shared_out = pl.pallas_call(
        _shared_body,
        grid=(n_s,),
        in_specs=[
            pl.BlockSpec((t, _D), lambda s: (0, 0)),
            pl.BlockSpec((_SCH, _D), lambda s: (s, 0)),
            pl.BlockSpec((_SCH, _D), lambda s: (s, 0)),
            pl.BlockSpec((_D, _SCH), lambda s: (0, s)),
            pl.BlockSpec((1, _D), lambda s: (0, 0)),
        ],
        out_specs=pl.BlockSpec((t, _D), lambda s: (0, 0)),
        out_shape=jax.ShapeDtypeStruct((t, _D), jnp.float32),
        scratch_shapes=[pltpu.VMEM((t, 1), jnp.float32)],
        compiler_params=pltpu.CompilerParams(
            vmem_limit_bytes=64 * 1024 * 1024),
    )(xb, shared_gate_w, shared_up_w, shared_down_w, shared_expert_gate_w)

    p1, p2, w1, w2, texp, tval = pl.pallas_call(
        _routing_body,
        out_shape=(
            jax.ShapeDtypeStruct((t, 1), jnp.int32),
            jax.ShapeDtypeStruct((t, 1), jnp.int32),
            jax.ShapeDtypeStruct((t, 1), jnp.float32),
            jax.ShapeDtypeStruct((t, 1), jnp.float32),
            jax.ShapeDtypeStruct((1, _MAXT), jnp.int32),
            jax.ShapeDtypeStruct((1, _MAXT), jnp.int32),
        ),
        compiler_params=pltpu.CompilerParams(
            vmem_limit_bytes=64 * 1024 * 1024),
    )(x, gate_w)

    pc = jnp.concatenate([p1.reshape(1, t), p2.reshape(1, t)], axis=1)
    xbi = jax.lax.bitcast_convert_type(
        xb.reshape(t, d // 2, 2), jnp.int32)
    x2 = jnp.concatenate([xbi, xbi], axis=0)
    xgi = _sc_scatter_rows(x2, pc, _MAXR)
    xg = jax.lax.bitcast_convert_type(
        xgi, jnp.bfloat16).reshape(_MAXR, d)

    gu_spec = pltpu.PrefetchScalarGridSpec(
        num_scalar_prefetch=2,
        grid=(_MAXT,),
        in_specs=[
            pl.BlockSpec((_TILE, _D), lambda j, te, tv: (j, 0)),
            pl.BlockSpec((1, _DFF, _D), lambda j, te, tv: (te[j], 0, 0)),
            pl.BlockSpec((1, _DFF, _D), lambda j, te, tv: (te[j], 0, 0)),
        ],
        out_specs=pl.BlockSpec((_TILE, _DFF), lambda j, te, tv: (j, 0)),
    )
    h = pl.pallas_call(
        _gu_body,
        grid_spec=gu_spec,
        out_shape=jax.ShapeDtypeStruct((_MAXR, _DFF), jnp.bfloat16),
        compiler_params=pltpu.CompilerParams(
            vmem_limit_bytes=64 * 1024 * 1024),
    )(texp.reshape(_MAXT), tval.reshape(_MAXT), xg,
      expert_gate_w, expert_up_w)

    dn_spec = pltpu.PrefetchScalarGridSpec(
        num_scalar_prefetch=2,
        grid=(_MAXT,),
        in_specs=[
            pl.BlockSpec((_TILE, _DFF), lambda j, te, tv: (j, 0)),
            pl.BlockSpec((1, _D, _DFF), lambda j, te, tv: (te[j], 0, 0)),
        ],
        out_specs=pl.BlockSpec((_TILE, _D), lambda j, te, tv: (j, 0)),
    )
    y = pl.pallas_call(
        _dn_body,
        grid_spec=dn_spec,
        out_shape=jax.ShapeDtypeStruct((_MAXR, _D), jnp.bfloat16),
        compiler_params=pltpu.CompilerParams(
            vmem_limit_bytes=64 * 1024 * 1024),
    )(texp.reshape(_MAXT), tval.reshape(_MAXT), h, expert_down_w)

    yi = jax.lax.bitcast_convert_type(
        y.reshape(_MAXR, _D // 2, 2), jnp.int32)
    ygi = _sc_gather_rows(yi, pc)
    yg = jax.lax.bitcast_convert_type(
        ygi, jnp.bfloat16).reshape(2 * t, d)

    nco = 4
    out = pl.pallas_call(
        _combine_body,
        grid=(nco,),
        in_specs=[
            pl.BlockSpec((t // nco, _D), lambda i: (i, 0)),
            pl.BlockSpec((t // nco, _D), lambda i: (i + nco, 0)),
            pl.BlockSpec((t // nco, 1), lambda i: (i, 0)),
            pl.BlockSpec((t // nco, 1), lambda i: (i, 0)),
            pl.BlockSpec((t // nco, _D), lambda i: (i, 0)),
        ],
        out_specs=pl.BlockSpec((t // nco, _D), lambda i: (i, 0)),
        out_shape=jax.ShapeDtypeStruct((t, _D), jnp.float32),
    )(yg, yg, w1, w2, shared_out)

    return out.reshape(b, seq, d)


# speedup vs baseline: 1.9521x; 1.9521x over previous
import jax
import jax.numpy as jnp
from jax.experimental import pallas as pl
from jax.experimental.pallas import tpu as pltpu

_E = 8
_D = 2048
_DFF = 1408
_SFF = 5632

_FCH = 352
_DCH = 256
_SCH = 256
_TCH = 512


def _router_body(xt_ref, gw_ref, ewt_ref):
    xt = xt_ref[...]
    gw = gw_ref[...]
    logits = jax.lax.dot_general(
        gw, xt, (((1,), (0,)), ((), ())), preferred_element_type=jnp.float32)
    iota = jax.lax.broadcasted_iota(jnp.int32, logits.shape, 0)
    m1 = jnp.max(logits, axis=0, keepdims=True)
    i1 = jnp.min(jnp.where(logits == m1, iota, _E), axis=0, keepdims=True)
    oh1 = iota == i1
    l2 = jnp.where(oh1, -jnp.inf, logits)
    m2 = jnp.max(l2, axis=0, keepdims=True)
    i2 = jnp.min(jnp.where(l2 == m2, iota, _E), axis=0, keepdims=True)
    oh2 = iota == i2
    w1 = 1.0 / (1.0 + jnp.exp(m2 - m1))
    w2 = 1.0 - w1
    ewt_ref[...] = jnp.where(oh1, w1, 0.0) + jnp.where(oh2, w2, 0.0)


def _moe_body(xt_ref, ewt_ref, ge_ref, ue_ref, de_ref, out_ref, ht_ref):
    e = pl.program_id(0)
    p = pl.program_id(1)
    xt = xt_ref[...]

    @pl.when(p < 4)
    def _():
        ge = ge_ref[0].astype(jnp.bfloat16)
        ue = ue_ref[0].astype(jnp.bfloat16)
        g = jax.lax.dot_general(
            ge, xt, (((1,), (0,)), ((), ())),
            preferred_element_type=jnp.float32)
        u = jax.lax.dot_general(
            ue, xt, (((1,), (0,)), ((), ())),
            preferred_element_type=jnp.float32)
        w = ewt_ref[0]
        ht = (g * jax.nn.sigmoid(g) * u * w).astype(jnp.bfloat16)
        row = pl.multiple_of(p * _FCH, 32)
        ht_ref[pl.ds(row, _FCH), :] = ht

    @pl.when(p >= 4)
    def _():
        de = de_ref[0].astype(jnp.bfloat16)
        tmp = jax.lax.dot_general(
            de, ht_ref[...], (((1,), (0,)), ((), ())),
            preferred_element_type=jnp.float32)
        row = pl.multiple_of((p - 4) * _DCH, _DCH)

        @pl.when(e == 0)
        def _():
            out_ref[pl.ds(row, _DCH), :] = tmp

        @pl.when(e > 0)
        def _():
            out_ref[pl.ds(row, _DCH), :] += tmp


def _shared_body(xb_ref, wg_ref, wu_ref, wd_ref, wsg_ref, out_ref, sig_ref):
    s = pl.program_id(0)
    xb = xb_ref[...]

    @pl.when(s == 0)
    def _():
        xf = xb.astype(jnp.float32)
        logit = jnp.sum(xf * wsg_ref[...], axis=1, keepdims=True)
        sig_ref[...] = jax.nn.sigmoid(logit)

    wg = wg_ref[...].astype(jnp.bfloat16)
    wu = wu_ref[...].astype(jnp.bfloat16)
    wd = wd_ref[...].astype(jnp.bfloat16)
    g = jax.lax.dot_general(
        xb, wg, (((1,), (1,)), ((), ())), preferred_element_type=jnp.float32)
    u = jax.lax.dot_general(
        xb, wu, (((1,), (1,)), ((), ())), preferred_element_type=jnp.float32)
    h = (g * jax.nn.sigmoid(g) * u * sig_ref[...]).astype(jnp.bfloat16)
    for k in range(4):
        hk = h[k * _TCH:(k + 1) * _TCH, :]
        tmp = jax.lax.dot_general(
            hk, wd, (((1,), (1,)), ((), ())),
            preferred_element_type=jnp.float32)

        @pl.when(s == 0)
        def _():
            out_ref[k * _TCH:(k + 1) * _TCH, :] = tmp

        @pl.when(s > 0)
        def _():
            out_ref[k * _TCH:(k + 1) * _TCH, :] += tmp


def kernel(hidden_states, gate_w, expert_gate_w, expert_up_w, expert_down_w,
           shared_gate_w, shared_up_w, shared_down_w, shared_expert_gate_w):
    b, seq, d = hidden_states.shape
    t = b * seq
    x = hidden_states.reshape(t, d)
    xt32 = jnp.swapaxes(x, 0, 1)
    xt = xt32.astype(jnp.bfloat16)
    xb = x.astype(jnp.bfloat16)

    ewt = pl.pallas_call(
        _router_body,
        out_shape=jax.ShapeDtypeStruct((_E, t), jnp.float32),
    )(xt32, gate_w)
    ewt3 = ewt.reshape(_E, 1, t)

    out_t = pl.pallas_call(
        _moe_body,
        grid=(_E, 12),
        in_specs=[
            pl.BlockSpec((_D, t), lambda e, p: (0, 0)),
            pl.BlockSpec((1, 1, t), lambda e, p: (e, 0, 0)),
            pl.BlockSpec((1, _FCH, _D),
                         lambda e, p: (e, jnp.minimum(p, 3), 0)),
            pl.BlockSpec((1, _FCH, _D),
                         lambda e, p: (e, jnp.minimum(p, 3), 0)),
            pl.BlockSpec((1, _DCH, _DFF),
                         lambda e, p: (e, jnp.maximum(p - 4, 0), 0)),
        ],
        out_specs=pl.BlockSpec((_D, t), lambda e, p: (0, 0)),
        out_shape=jax.ShapeDtypeStruct((_D, t), jnp.float32),
        scratch_shapes=[pltpu.VMEM((_DFF, t), jnp.bfloat16)],
        compiler_params=pltpu.CompilerParams(
            vmem_limit_bytes=64 * 1024 * 1024),
    )(xt, ewt3, expert_gate_w, expert_up_w, expert_down_w)

    n_s = _SFF // _SCH
    shared_out = pl.pallas_call(
        _shared_body,
        grid=(n_s,),
        in_specs=[
            pl.BlockSpec((t, _D), lambda s: (0, 0)),
            pl.BlockSpec((_SCH, _D), lambda s: (s, 0)),
            pl.BlockSpec((_SCH, _D), lambda s: (s, 0)),
            pl.BlockSpec((_D, _SCH), lambda s: (0, s)),
            pl.BlockSpec((1, _D), lambda s: (0, 0)),
        ],
        out_specs=pl.BlockSpec((t, _D), lambda s: (0, 0)),
        out_shape=jax.ShapeDtypeStruct((t, _D), jnp.float32),
        scratch_shapes=[pltpu.VMEM((t, 1), jnp.float32)],
        compiler_params=pltpu.CompilerParams(
            vmem_limit_bytes=64 * 1024 * 1024),
    )(xb, shared_gate_w, shared_up_w, shared_down_w, shared_expert_gate_w)

    return (jnp.swapaxes(out_t, 0, 1) + shared_out).reshape(b, seq, d)
